# baseline (speedup 1.0000x reference)
import functools

import jax
import jax.numpy as jnp
from jax import lax
from jax.experimental import pallas as pl
from jax.experimental.pallas import tpu as pltpu
from jax.experimental.pallas import tpu_sc as plsc

N = 10000
E = 160000
CH = 128
PADE = 163840
NACC = N + 16
BLK = 2000

_MESH = plsc.VectorSubcoreMesh(core_axis_name="c", subcore_axis_name="s")


def _deg_body(dst_hbm, zeros_hbm, ones_hbm, out_hbm,
              acc, ones_v, dbuf, sa, sb):
    c = lax.axis_index("c")
    s = lax.axis_index("s")
    @pl.when(s < 10)
    def _():
        pltpu.sync_copy(zeros_hbm.at[pl.ds(s * 1024, 1024)],
                        acc.at[pl.ds(s * 1024, 1024)])
    pltpu.sync_copy(ones_hbm, ones_v)
    pltpu.sync_copy(dst_hbm.at[pl.ds(c * 640 + s * 40, 40)], dbuf)
    plsc.subcore_barrier()

    def _sadd(j, sem):
        pltpu.async_copy(ones_v, acc.at[dbuf.at[j]], sem, add=True)

    def _swait(j, sem):
        pltpu.make_async_copy(ones_v, acc.at[dbuf.at[j]], sem).wait()

    _sadd(0, sa)

    def body(k2, carry):
        a = 2 * k2

        @pl.when(k2 > 0)
        def _():
            _swait(a - 1, sb)
        _sadd(a + 1, sb)
        _swait(a, sa)

        @pl.when(k2 < 19)
        def _():
            _sadd(a + 2, sa)
        return carry

    lax.fori_loop(0, 20, body, 0)
    _swait(39, sb)
    plsc.subcore_barrier()

    @pl.when(s < 10)
    def _():
        pltpu.sync_copy(acc.at[pl.ds(s * 1024, 1024)],
                        out_hbm.at[pl.ds(c * 10240 + s * 1024, 1024)])


@functools.partial(
    pl.kernel,
    out_type=jax.ShapeDtypeStruct((20480,), jnp.float32),
    mesh=_MESH,
    scratch_types=[
        pltpu.VMEM_SHARED((10240,), jnp.float32),
        pltpu.VMEM((CH,), jnp.float32),
        pltpu.VMEM((40, CH), jnp.int32),
        pltpu.SemaphoreType.DMA,
        pltpu.SemaphoreType.DMA,
    ],
)
def _deg_sc(dst_hbm, zeros_hbm, ones_hbm, out_hbm, *scratch):
    _deg_body(dst_hbm, zeros_hbm, ones_hbm, out_hbm, *scratch)


def _edge_pipeline(gflat_hbm, acc, sbuf, dbuf, nchunks,
                   ra, rb, gsa, gsb, ssa, ssb):
    def _gstart(rows_ref, sem, j):
        pltpu.async_copy(gflat_hbm.at[sbuf.at[j]], rows_ref, sem)

    def _gwait(rows_ref, sem, j):
        pltpu.make_async_copy(gflat_hbm.at[sbuf.at[j]], rows_ref, sem).wait()

    def _sstart(rows_ref, sem, j):
        pltpu.async_copy(rows_ref, acc.at[dbuf.at[j]], sem, add=True)

    def _swait(rows_ref, sem, j):
        pltpu.make_async_copy(rows_ref, acc.at[dbuf.at[j]], sem).wait()

    _gstart(ra, gsa, 0)

    def body(k2, carry):
        a = 2 * k2

        @pl.when(k2 > 0)
        def _():
            _swait(rb, ssb, a - 1)
        _gstart(rb, gsb, a + 1)
        _gwait(ra, gsa, a)
        _sstart(ra, ssa, a)
        _gwait(rb, gsb, a + 1)
        _swait(ra, ssa, a)

        @pl.when(k2 < nchunks // 2 - 1)
        def _():
            _gstart(ra, gsa, a + 2)
        _sstart(rb, ssb, a + 1)
        return carry

    lax.fori_loop(0, nchunks // 2, body, 0)
    _swait(rb, ssb, nchunks - 1)


def _agg_edge_body(gflat_hbm, srcflat_hbm, dst_hbm, out_hbm,
                   acc, sbuf, dbuf, *bufs):
    c = lax.axis_index("c")
    s = lax.axis_index("s")

    @pl.when(s < 5)
    def _():
        pltpu.sync_copy(gflat_hbm.at[pl.ds(s * 2000, 2000)],
                        acc.at[pl.ds(s * 2000, 2000)])
    base = c * 640 + s * 40
    pltpu.sync_copy(srcflat_hbm.at[pl.ds(base, 40)], sbuf)
    pltpu.sync_copy(dst_hbm.at[pl.ds(base, 40)], dbuf)
    plsc.subcore_barrier()

    _edge_pipeline(gflat_hbm, acc, sbuf, dbuf, 40, *bufs)
    plsc.subcore_barrier()

    @pl.when(s < 5)
    def _():
        pltpu.sync_copy(acc.at[pl.ds(s * 2000, 2000)],
                        out_hbm.at[c, pl.ds(s * 2000, 2000)])


def _agg_scratch(row_shape, dtype):
    return [
        pltpu.VMEM_SHARED((NACC,) + row_shape, dtype),
        pltpu.VMEM((40, CH), jnp.int32),
        pltpu.VMEM((40, CH), jnp.int32),
        pltpu.VMEM((CH,) + row_shape, dtype),
        pltpu.VMEM((CH,) + row_shape, dtype),
        pltpu.SemaphoreType.DMA,
        pltpu.SemaphoreType.DMA,
        pltpu.SemaphoreType.DMA,
        pltpu.SemaphoreType.DMA,
    ]


_agg_bf16 = functools.partial(
    pl.kernel,
    out_type=jax.ShapeDtypeStruct((2, N, 256), jnp.bfloat16),
    mesh=_MESH,
    scratch_types=_agg_scratch((256,), jnp.bfloat16),
    compiler_params=pltpu.CompilerParams(use_tc_tiling_on_sc=False),
)(_agg_edge_body)

_agg_f32 = functools.partial(
    pl.kernel,
    out_type=jax.ShapeDtypeStruct((2, N, 128), jnp.float32),
    mesh=_MESH,
    scratch_types=_agg_scratch((128,), jnp.float32),
)(_agg_edge_body)


def _leaky(z):
    return jnp.where(z >= 0, z, z * 0.01)


def _dinv(degp_ref):
    return lax.rsqrt(degp_ref[:, 0:1] + degp_ref[:, 1:2] + 1.0)


def _first_body(x_ref, w_ref, o_ref):
    o_ref[...] = jnp.dot(x_ref[...], w_ref[...],
                         preferred_element_type=jnp.float32)


def _first_tc(x, W):
    d = W.shape[1]
    return pl.pallas_call(
        _first_body,
        grid=(N // BLK,),
        in_specs=[pl.BlockSpec((BLK, x.shape[1]), lambda i: (i, 0)),
                  pl.BlockSpec(W.shape, lambda i: (0, 0))],
        out_specs=pl.BlockSpec((BLK, d), lambda i: (i, 0)),
        out_shape=jax.ShapeDtypeStruct((N, d), jnp.float32),
    )(x, W)


def _mid_body(s_ref, w_ref, degp_ref, b_ref, o_ref):
    h = _leaky(_dinv(degp_ref) * s_ref[...] + b_ref[...])
    o_ref[...] = jnp.dot(h, w_ref[...], preferred_element_type=jnp.float32)


def _mid_tc(s, W, degp, b):
    d = W.shape[1]
    dp = s.shape[1]
    return pl.pallas_call(
        _mid_body,
        grid=(N // BLK,),
        in_specs=[pl.BlockSpec((BLK, dp), lambda i: (i, 0)),
                  pl.BlockSpec(W.shape, lambda i: (0, 0)),
                  pl.BlockSpec((BLK, 2), lambda i: (i, 0)),
                  pl.BlockSpec((1, dp), lambda i: (0, 0))],
        out_specs=pl.BlockSpec((BLK, d), lambda i: (i, 0)),
        out_shape=jax.ShapeDtypeStruct((N, d), jnp.float32),
    )(s, W, degp, b)


def _last_body(p_ref, g_ref, degp_ref, b_ref, o_ref):
    sagg = p_ref[0] + p_ref[1] - g_ref[...]
    o_ref[...] = _leaky(_dinv(degp_ref) * sagg + b_ref[...])


def _last_tc(p, g, degp, b):
    d = g.shape[1]
    return pl.pallas_call(
        _last_body,
        grid=(N // BLK,),
        in_specs=[pl.BlockSpec((2, BLK, d), lambda i: (0, i, 0)),
                  pl.BlockSpec((BLK, d), lambda i: (i, 0)),
                  pl.BlockSpec((BLK, 2), lambda i: (i, 0)),
                  pl.BlockSpec((1, d), lambda i: (0, 0))],
        out_specs=pl.BlockSpec((BLK, d), lambda i: (i, 0)),
        out_shape=jax.ShapeDtypeStruct((N, d), jnp.float32),
    )(p, g, degp, b)


def kernel(x, edge_index, W1, b1, W2, b2, W3, b3):
    src = edge_index[0]
    dst = edge_index[1]
    pad = PADE - E
    ar = jnp.arange(pad, dtype=jnp.int32)
    srcflat = jnp.concatenate([src, ar % 64]).reshape(1280, CH)
    dstp = jnp.concatenate([dst, N + (ar % 16)]).reshape(1280, CH)

    degp = _deg_sc(dstp, jnp.zeros((10240,), jnp.float32),
                   jnp.ones((CH,), jnp.float32))
    degp = degp.reshape(2, 10240)[:, :N].T
    dinv = lax.rsqrt(degp[:, 0:1] + degp[:, 1:2] + 1.0)

    t = _first_tc(x, W1)
    g = (dinv * t).astype(jnp.bfloat16)
    p = _agg_bf16(g, srcflat, dstp)
    s = (p[0].astype(jnp.float32) + p[1].astype(jnp.float32)
         - g.astype(jnp.float32))
    t = _mid_tc(s, W2, degp, b1.reshape(1, -1))
    g = (dinv * t).astype(jnp.bfloat16)
    p = _agg_bf16(g, srcflat, dstp)
    s = (p[0].astype(jnp.float32) + p[1].astype(jnp.float32)
         - g.astype(jnp.float32))
    t = _mid_tc(s, W3, degp, b2.reshape(1, -1))
    g = dinv * t
    p = _agg_f32(g, srcflat, dstp)
    return _last_tc(p, g, degp, b3.reshape(1, -1))


# speedup vs baseline: 1.0478x; 1.0478x over previous
import functools

import jax
import jax.numpy as jnp
from jax import lax
from jax.experimental import pallas as pl
from jax.experimental.pallas import tpu as pltpu
from jax.experimental.pallas import tpu_sc as plsc

N = 10000
E = 160000
CH = 128
PADE = 163840
NACC = N + 16
BLK = 2000

_MESH = plsc.VectorSubcoreMesh(core_axis_name="c", subcore_axis_name="s")


def _deg_body(dst_hbm, zeros_hbm, ones_hbm, out_hbm,
              acc, ones_v, dbuf, sa, sb):
    c = lax.axis_index("c")
    s = lax.axis_index("s")
    @pl.when(s < 10)
    def _():
        pltpu.sync_copy(zeros_hbm.at[pl.ds(s * 1024, 1024)],
                        acc.at[pl.ds(s * 1024, 1024)])
    pltpu.sync_copy(ones_hbm, ones_v)
    pltpu.sync_copy(dst_hbm.at[pl.ds(c * 640 + s * 40, 40)], dbuf)
    plsc.subcore_barrier()

    def _sadd(j, sem):
        pltpu.async_copy(ones_v, acc.at[dbuf.at[j]], sem, add=True)

    def _swait(j, sem):
        pltpu.make_async_copy(ones_v, acc.at[dbuf.at[j]], sem).wait()

    _sadd(0, sa)

    def body(k2, carry):
        a = 2 * k2

        @pl.when(k2 > 0)
        def _():
            _swait(a - 1, sb)
        _sadd(a + 1, sb)
        _swait(a, sa)

        @pl.when(k2 < 19)
        def _():
            _sadd(a + 2, sa)
        return carry

    lax.fori_loop(0, 20, body, 0)
    _swait(39, sb)
    plsc.subcore_barrier()

    @pl.when(s < 10)
    def _():
        pltpu.sync_copy(acc.at[pl.ds(s * 1024, 1024)],
                        out_hbm.at[pl.ds(c * 10240 + s * 1024, 1024)])


@functools.partial(
    pl.kernel,
    out_type=jax.ShapeDtypeStruct((20480,), jnp.float32),
    mesh=_MESH,
    scratch_types=[
        pltpu.VMEM_SHARED((10240,), jnp.float32),
        pltpu.VMEM((CH,), jnp.float32),
        pltpu.VMEM((40, CH), jnp.int32),
        pltpu.SemaphoreType.DMA,
        pltpu.SemaphoreType.DMA,
    ],
)
def _deg_sc(dst_hbm, zeros_hbm, ones_hbm, out_hbm, *scratch):
    _deg_body(dst_hbm, zeros_hbm, ones_hbm, out_hbm, *scratch)


def _edge_pipeline(gflat_hbm, acc, sbuf, dbuf, nchunks,
                   ra, rb, gsa, gsb, ssa, ssb):
    def _gstart(rows_ref, sem, j):
        pltpu.async_copy(gflat_hbm.at[sbuf.at[j]], rows_ref, sem)

    def _gwait(rows_ref, sem, j):
        pltpu.make_async_copy(gflat_hbm.at[sbuf.at[j]], rows_ref, sem).wait()

    def _sstart(rows_ref, sem, j):
        pltpu.async_copy(rows_ref, acc.at[dbuf.at[j]], sem, add=True)

    def _swait(rows_ref, sem, j):
        pltpu.make_async_copy(rows_ref, acc.at[dbuf.at[j]], sem).wait()

    _gstart(ra, gsa, 0)

    def body(k2, carry):
        a = 2 * k2

        @pl.when(k2 > 0)
        def _():
            _swait(rb, ssb, a - 1)
        _gstart(rb, gsb, a + 1)
        _gwait(ra, gsa, a)
        _sstart(ra, ssa, a)
        _gwait(rb, gsb, a + 1)
        _swait(ra, ssa, a)

        @pl.when(k2 < nchunks // 2 - 1)
        def _():
            _gstart(ra, gsa, a + 2)
        _sstart(rb, ssb, a + 1)
        return carry

    lax.fori_loop(0, nchunks // 2, body, 0)
    _swait(rb, ssb, nchunks - 1)


def _agg_edge_body(gflat_hbm, srcflat_hbm, dst_hbm, out_hbm,
                   acc, sbuf, dbuf, *bufs):
    c = lax.axis_index("c")
    s = lax.axis_index("s")

    @pl.when(s < 5)
    def _():
        pltpu.sync_copy(gflat_hbm.at[pl.ds(s * 2000, 2000)],
                        acc.at[pl.ds(s * 2000, 2000)])
    base = c * 640 + s * 40
    pltpu.sync_copy(srcflat_hbm.at[pl.ds(base, 40)], sbuf)
    pltpu.sync_copy(dst_hbm.at[pl.ds(base, 40)], dbuf)
    plsc.subcore_barrier()

    _edge_pipeline(gflat_hbm, acc, sbuf, dbuf, 40, *bufs)
    plsc.subcore_barrier()

    @pl.when(s < 5)
    def _():
        pltpu.sync_copy(acc.at[pl.ds(s * 2000, 2000)],
                        out_hbm.at[c, pl.ds(s * 2000, 2000)])


def _agg_scratch(row_shape, dtype):
    return [
        pltpu.VMEM_SHARED((NACC,) + row_shape, dtype),
        pltpu.VMEM((40, CH), jnp.int32),
        pltpu.VMEM((40, CH), jnp.int32),
        pltpu.VMEM((CH,) + row_shape, dtype),
        pltpu.VMEM((CH,) + row_shape, dtype),
        pltpu.SemaphoreType.DMA,
        pltpu.SemaphoreType.DMA,
        pltpu.SemaphoreType.DMA,
        pltpu.SemaphoreType.DMA,
    ]


_agg_bf16 = functools.partial(
    pl.kernel,
    out_type=jax.ShapeDtypeStruct((2, N, 256), jnp.bfloat16),
    mesh=_MESH,
    scratch_types=_agg_scratch((256,), jnp.bfloat16),
    compiler_params=pltpu.CompilerParams(use_tc_tiling_on_sc=False),
)(_agg_edge_body)

_agg_f32 = functools.partial(
    pl.kernel,
    out_type=jax.ShapeDtypeStruct((2, N, 128), jnp.float32),
    mesh=_MESH,
    scratch_types=_agg_scratch((128,), jnp.float32),
)(_agg_edge_body)


def _leaky(z):
    return jnp.where(z >= 0, z, z * 0.01)


def _dinv(degp_ref):
    return lax.rsqrt(degp_ref[:, 0:1] + degp_ref[:, 1:2] + 1.0)


def _first_body(x_ref, w_ref, degp_ref, o_ref):
    t = jnp.dot(x_ref[...], w_ref[...], preferred_element_type=jnp.float32)
    o_ref[...] = (_dinv(degp_ref) * t).astype(o_ref.dtype)


def _first_tc(x, W, degp):
    d = W.shape[1]
    return pl.pallas_call(
        _first_body,
        grid=(N // BLK,),
        in_specs=[pl.BlockSpec((BLK, x.shape[1]), lambda i: (i, 0)),
                  pl.BlockSpec(W.shape, lambda i: (0, 0)),
                  pl.BlockSpec((BLK, 2), lambda i: (i, 0))],
        out_specs=pl.BlockSpec((BLK, d), lambda i: (i, 0)),
        out_shape=jax.ShapeDtypeStruct((N, d), jnp.bfloat16),
    )(x, W, degp)


def _mid_body(p_ref, g_ref, w_ref, degp_ref, b_ref, o_ref):
    dinv = _dinv(degp_ref)
    sagg = (p_ref[0].astype(jnp.float32) + p_ref[1].astype(jnp.float32)
            - g_ref[...].astype(jnp.float32))
    h = _leaky(dinv * sagg + b_ref[...])
    t = jnp.dot(h, w_ref[...], preferred_element_type=jnp.float32)
    o_ref[...] = (dinv * t).astype(o_ref.dtype)


def _mid_tc(p, g, W, degp, b, out_dtype):
    d = W.shape[1]
    dp = g.shape[1]
    return pl.pallas_call(
        _mid_body,
        grid=(N // BLK,),
        in_specs=[pl.BlockSpec((2, BLK, dp), lambda i: (0, i, 0)),
                  pl.BlockSpec((BLK, dp), lambda i: (i, 0)),
                  pl.BlockSpec(W.shape, lambda i: (0, 0)),
                  pl.BlockSpec((BLK, 2), lambda i: (i, 0)),
                  pl.BlockSpec((1, dp), lambda i: (0, 0))],
        out_specs=pl.BlockSpec((BLK, d), lambda i: (i, 0)),
        out_shape=jax.ShapeDtypeStruct((N, d), out_dtype),
    )(p, g, W, degp, b)


def _last_body(p_ref, g_ref, degp_ref, b_ref, o_ref):
    sagg = p_ref[0] + p_ref[1] - g_ref[...]
    o_ref[...] = _leaky(_dinv(degp_ref) * sagg + b_ref[...])


def _last_tc(p, g, degp, b):
    d = g.shape[1]
    return pl.pallas_call(
        _last_body,
        grid=(N // BLK,),
        in_specs=[pl.BlockSpec((2, BLK, d), lambda i: (0, i, 0)),
                  pl.BlockSpec((BLK, d), lambda i: (i, 0)),
                  pl.BlockSpec((BLK, 2), lambda i: (i, 0)),
                  pl.BlockSpec((1, d), lambda i: (0, 0))],
        out_specs=pl.BlockSpec((BLK, d), lambda i: (i, 0)),
        out_shape=jax.ShapeDtypeStruct((N, d), jnp.float32),
    )(p, g, degp, b)


def kernel(x, edge_index, W1, b1, W2, b2, W3, b3):
    src = edge_index[0]
    dst = edge_index[1]
    pad = PADE - E
    ar = jnp.arange(pad, dtype=jnp.int32)
    srcflat = jnp.concatenate([src, ar % 64]).reshape(1280, CH)
    dstp = jnp.concatenate([dst, N + (ar % 16)]).reshape(1280, CH)

    degp = _deg_sc(dstp, jnp.zeros((10240,), jnp.float32),
                   jnp.ones((CH,), jnp.float32))
    degp = degp.reshape(2, 10240)[:, :N].T

    g = _first_tc(x, W1, degp)
    p = _agg_bf16(g, srcflat, dstp)
    g = _mid_tc(p, g, W2, degp,
                b1.reshape(1, -1), jnp.bfloat16)
    p = _agg_bf16(g, srcflat, dstp)
    g = _mid_tc(p, g, W3, degp,
                b2.reshape(1, -1), jnp.float32)
    p = _agg_f32(g, srcflat, dstp)
    return _last_tc(p, g, degp, b3.reshape(1, -1))


# speedup vs baseline: 1.0480x; 1.0002x over previous
import functools

import jax
import jax.numpy as jnp
from jax import lax
from jax.experimental import pallas as pl
from jax.experimental.pallas import tpu as pltpu
from jax.experimental.pallas import tpu_sc as plsc

N = 10000
E = 160000
CH = 128
PADE = 163840
NACC = N + 16
BLK = 2000

_MESH = plsc.VectorSubcoreMesh(core_axis_name="c", subcore_axis_name="s")


def _deg_body(dst_hbm, zeros_hbm, ones_hbm, out_hbm,
              acc, ones_v, dbuf, sa, sb):
    c = lax.axis_index("c")
    s = lax.axis_index("s")
    @pl.when(s < 10)
    def _():
        pltpu.sync_copy(zeros_hbm.at[pl.ds(s * 1024, 1024)],
                        acc.at[pl.ds(s * 1024, 1024)])
    pltpu.sync_copy(ones_hbm, ones_v)
    pltpu.sync_copy(dst_hbm.at[pl.ds(c * 640 + s * 40, 40)], dbuf)
    plsc.subcore_barrier()

    def _sadd(j, sem):
        pltpu.async_copy(ones_v, acc.at[dbuf.at[j]], sem, add=True)

    def _swait(j, sem):
        pltpu.make_async_copy(ones_v, acc.at[dbuf.at[j]], sem).wait()

    _sadd(0, sa)

    def body(k2, carry):
        a = 2 * k2

        @pl.when(k2 > 0)
        def _():
            _swait(a - 1, sb)
        _sadd(a + 1, sb)
        _swait(a, sa)

        @pl.when(k2 < 19)
        def _():
            _sadd(a + 2, sa)
        return carry

    lax.fori_loop(0, 20, body, 0)
    _swait(39, sb)
    plsc.subcore_barrier()

    @pl.when(s < 10)
    def _():
        pltpu.sync_copy(acc.at[pl.ds(s * 1024, 1024)],
                        out_hbm.at[pl.ds(c * 10240 + s * 1024, 1024)])


@functools.partial(
    pl.kernel,
    out_type=jax.ShapeDtypeStruct((20480,), jnp.float32),
    mesh=_MESH,
    scratch_types=[
        pltpu.VMEM_SHARED((10240,), jnp.float32),
        pltpu.VMEM((CH,), jnp.float32),
        pltpu.VMEM((40, CH), jnp.int32),
        pltpu.SemaphoreType.DMA,
        pltpu.SemaphoreType.DMA,
    ],
)
def _deg_sc(dst_hbm, zeros_hbm, ones_hbm, out_hbm, *scratch):
    _deg_body(dst_hbm, zeros_hbm, ones_hbm, out_hbm, *scratch)


def _edge_pipeline(gflat_hbm, acc, sbuf, dbuf, nchunks,
                   ra, rb, gsa, gsb, ssa, ssb):
    def _gstart(rows_ref, sem, j):
        pltpu.async_copy(gflat_hbm.at[sbuf.at[j]], rows_ref, sem)

    def _gwait(rows_ref, sem, j):
        pltpu.make_async_copy(gflat_hbm.at[sbuf.at[j]], rows_ref, sem).wait()

    def _sstart(rows_ref, sem, j):
        pltpu.async_copy(rows_ref, acc.at[dbuf.at[j]], sem, add=True)

    def _swait(rows_ref, sem, j):
        pltpu.make_async_copy(rows_ref, acc.at[dbuf.at[j]], sem).wait()

    _gstart(ra, gsa, 0)

    def body(k2, carry):
        a = 2 * k2

        @pl.when(k2 > 0)
        def _():
            _swait(rb, ssb, a - 1)
        _gstart(rb, gsb, a + 1)
        _gwait(ra, gsa, a)
        _sstart(ra, ssa, a)
        _gwait(rb, gsb, a + 1)
        _swait(ra, ssa, a)

        @pl.when(k2 < nchunks // 2 - 1)
        def _():
            _gstart(ra, gsa, a + 2)
        _sstart(rb, ssb, a + 1)
        return carry

    lax.fori_loop(0, nchunks // 2, body, 0)
    _swait(rb, ssb, nchunks - 1)


def _agg_edge_body(gflat_hbm, srcflat_hbm, dst_hbm, out_hbm,
                   acc, sbuf, dbuf, *bufs):
    c = lax.axis_index("c")
    s = lax.axis_index("s")

    @pl.when(s < 5)
    def _():
        pltpu.sync_copy(gflat_hbm.at[pl.ds(s * 2000, 2000)],
                        acc.at[pl.ds(s * 2000, 2000)])
    base = c * 640 + s * 40
    pltpu.sync_copy(srcflat_hbm.at[pl.ds(base, 40)], sbuf)
    pltpu.sync_copy(dst_hbm.at[pl.ds(base, 40)], dbuf)
    plsc.subcore_barrier()

    _edge_pipeline(gflat_hbm, acc, sbuf, dbuf, 40, *bufs)
    plsc.subcore_barrier()

    @pl.when(s < 5)
    def _():
        pltpu.sync_copy(acc.at[pl.ds(s * 2000, 2000)],
                        out_hbm.at[c, pl.ds(s * 2000, 2000)])


def _agg_scratch(row_shape, dtype):
    return [
        pltpu.VMEM_SHARED((NACC,) + row_shape, dtype),
        pltpu.VMEM((40, CH), jnp.int32),
        pltpu.VMEM((40, CH), jnp.int32),
        pltpu.VMEM((CH,) + row_shape, dtype),
        pltpu.VMEM((CH,) + row_shape, dtype),
        pltpu.SemaphoreType.DMA,
        pltpu.SemaphoreType.DMA,
        pltpu.SemaphoreType.DMA,
        pltpu.SemaphoreType.DMA,
    ]


_agg_bf16 = functools.partial(
    pl.kernel,
    out_type=jax.ShapeDtypeStruct((2, N, 256), jnp.bfloat16),
    mesh=_MESH,
    scratch_types=_agg_scratch((256,), jnp.bfloat16),
    compiler_params=pltpu.CompilerParams(use_tc_tiling_on_sc=False,
                                         skip_device_barrier=True),
)(_agg_edge_body)

_agg_f32 = functools.partial(
    pl.kernel,
    out_type=jax.ShapeDtypeStruct((2, N, 128), jnp.float32),
    mesh=_MESH,
    scratch_types=_agg_scratch((128,), jnp.float32),
    compiler_params=pltpu.CompilerParams(skip_device_barrier=True),
)(_agg_edge_body)


def _leaky(z):
    return jnp.where(z >= 0, z, z * 0.01)


def _dinv(degp_ref):
    return lax.rsqrt(degp_ref[:, 0:1] + degp_ref[:, 1:2] + 1.0)


def _first_body(x_ref, w_ref, degp_ref, o_ref):
    t = jnp.dot(x_ref[...], w_ref[...], preferred_element_type=jnp.float32)
    o_ref[...] = (_dinv(degp_ref) * t).astype(o_ref.dtype)


def _first_tc(x, W, degp):
    d = W.shape[1]
    return pl.pallas_call(
        _first_body,
        grid=(N // BLK,),
        in_specs=[pl.BlockSpec((BLK, x.shape[1]), lambda i: (i, 0)),
                  pl.BlockSpec(W.shape, lambda i: (0, 0)),
                  pl.BlockSpec((BLK, 2), lambda i: (i, 0))],
        out_specs=pl.BlockSpec((BLK, d), lambda i: (i, 0)),
        out_shape=jax.ShapeDtypeStruct((N, d), jnp.bfloat16),
    )(x, W, degp)


def _mid_body(p_ref, g_ref, w_ref, degp_ref, b_ref, o_ref):
    dinv = _dinv(degp_ref)
    sagg = (p_ref[0].astype(jnp.float32) + p_ref[1].astype(jnp.float32)
            - g_ref[...].astype(jnp.float32))
    h = _leaky(dinv * sagg + b_ref[...])
    t = jnp.dot(h, w_ref[...], preferred_element_type=jnp.float32)
    o_ref[...] = (dinv * t).astype(o_ref.dtype)


def _mid_tc(p, g, W, degp, b, out_dtype):
    d = W.shape[1]
    dp = g.shape[1]
    return pl.pallas_call(
        _mid_body,
        grid=(N // BLK,),
        in_specs=[pl.BlockSpec((2, BLK, dp), lambda i: (0, i, 0)),
                  pl.BlockSpec((BLK, dp), lambda i: (i, 0)),
                  pl.BlockSpec(W.shape, lambda i: (0, 0)),
                  pl.BlockSpec((BLK, 2), lambda i: (i, 0)),
                  pl.BlockSpec((1, dp), lambda i: (0, 0))],
        out_specs=pl.BlockSpec((BLK, d), lambda i: (i, 0)),
        out_shape=jax.ShapeDtypeStruct((N, d), out_dtype),
    )(p, g, W, degp, b)


def _last_body(p_ref, g_ref, degp_ref, b_ref, o_ref):
    sagg = p_ref[0] + p_ref[1] - g_ref[...]
    o_ref[...] = _leaky(_dinv(degp_ref) * sagg + b_ref[...])


def _last_tc(p, g, degp, b):
    d = g.shape[1]
    return pl.pallas_call(
        _last_body,
        grid=(N // BLK,),
        in_specs=[pl.BlockSpec((2, BLK, d), lambda i: (0, i, 0)),
                  pl.BlockSpec((BLK, d), lambda i: (i, 0)),
                  pl.BlockSpec((BLK, 2), lambda i: (i, 0)),
                  pl.BlockSpec((1, d), lambda i: (0, 0))],
        out_specs=pl.BlockSpec((BLK, d), lambda i: (i, 0)),
        out_shape=jax.ShapeDtypeStruct((N, d), jnp.float32),
    )(p, g, degp, b)


def kernel(x, edge_index, W1, b1, W2, b2, W3, b3):
    src = edge_index[0]
    dst = edge_index[1]
    pad = PADE - E
    ar = jnp.arange(pad, dtype=jnp.int32)
    srcflat = jnp.concatenate([src, ar % 64]).reshape(1280, CH)
    dstp = jnp.concatenate([dst, N + (ar % 16)]).reshape(1280, CH)

    degp = _deg_sc(dstp, jnp.zeros((10240,), jnp.float32),
                   jnp.ones((CH,), jnp.float32))
    degp = degp.reshape(2, 10240)[:, :N].T

    g = _first_tc(x, W1, degp)
    p = _agg_bf16(g, srcflat, dstp)
    g = _mid_tc(p, g, W2, degp,
                b1.reshape(1, -1), jnp.bfloat16)
    p = _agg_bf16(g, srcflat, dstp)
    g = _mid_tc(p, g, W3, degp,
                b2.reshape(1, -1), jnp.float32)
    p = _agg_f32(g, srcflat, dstp)
    return _last_tc(p, g, degp, b3.reshape(1, -1))
